# baseline (speedup 1.0000x reference)
import jax
import jax.numpy as jnp
from jax.experimental import pallas as pl
from jax.experimental.pallas import tpu as pltpu


def _csum_kernel(c_ref, o_ref):
    o_ref[...] = jnp.sum(c_ref[...], axis=1)


def _unsup_embed_kernel(x_ref, c_ref, cs_ref, w_ref, b_ref, o_ref):
    C = c_ref.shape[0]
    x = x_ref[...]

    pooled = jnp.mean(x, axis=2)
    sel = jnp.tanh(
        jax.lax.dot_general(
            pooled, w_ref[...], (((1,), (1,)), ((), ())),
            preferred_element_type=jnp.float32,
        )
        + b_ref[...]
    )

    fs = jnp.sum(x, axis=1)
    cs = cs_ref[...]
    dist = jnp.sum(jnp.abs(fs[:, None, :] - cs[None, :, :]), axis=2)
    mins = jnp.min(dist, axis=1, keepdims=True)
    cidx = jax.lax.broadcasted_iota(jnp.int32, dist.shape, 1)
    first = jnp.min(jnp.where(dist == mins, cidx, C), axis=1, keepdims=True)

    sel3 = sel[:, :, None]
    BB = x.shape[0]
    for b in range(BB):
        lab = first[b, 0]
        o_ref[b] = x[b] + sel3[b] * c_ref[lab]


def kernel(feats, centroids, W_sel, b_sel):
    B, D, H, W = feats.shape
    C = centroids.shape[0]
    HW = H * W
    x = feats.reshape(B, D, HW)
    cents = centroids.reshape(C, D, HW)
    b2 = b_sel.reshape(1, D)

    csum = pl.pallas_call(
        _csum_kernel,
        out_shape=jax.ShapeDtypeStruct((C, HW), jnp.float32),
    )(cents)

    BB = 4
    out = pl.pallas_call(
        _unsup_embed_kernel,
        grid=(B // BB,),
        in_specs=[
            pl.BlockSpec((BB, D, HW), lambda i: (i, 0, 0)),
            pl.BlockSpec((C, D, HW), lambda i: (0, 0, 0)),
            pl.BlockSpec((C, HW), lambda i: (0, 0)),
            pl.BlockSpec((D, D), lambda i: (0, 0)),
            pl.BlockSpec((1, D), lambda i: (0, 0)),
        ],
        out_specs=pl.BlockSpec((BB, D, HW), lambda i: (i, 0, 0)),
        out_shape=jax.ShapeDtypeStruct((B, D, HW), jnp.float32),
        compiler_params=pltpu.CompilerParams(
            dimension_semantics=("parallel",),
        ),
    )(x, cents, csum, W_sel, b2)
    return out.reshape(B, D, H, W)


# speedup vs baseline: 2.4913x; 2.4913x over previous
import jax
import jax.numpy as jnp
from jax.experimental import pallas as pl
from jax.experimental.pallas import tpu as pltpu


def _csum_kernel(c_ref, o_ref):
    o_ref[...] = jnp.sum(c_ref[...], axis=2)


def _stats_kernel(x_ref, cs_ref, w_ref, b_ref, sel_ref, lab_ref):
    C = cs_ref.shape[1]
    x = x_ref[...]

    pooled = jnp.mean(x, axis=0)
    sel_ref[...] = jnp.tanh(
        jax.lax.dot_general(
            pooled, w_ref[...], (((1,), (1,)), ((), ())),
            preferred_element_type=jnp.float32,
        )
        + b_ref[...]
    )

    fs = jnp.sum(x, axis=2, keepdims=True)
    cs = cs_ref[...][:, None, :]
    dist = jnp.sum(jnp.abs(fs - cs), axis=0)
    mins = jnp.min(dist, axis=1, keepdims=True)
    cidx = jax.lax.broadcasted_iota(jnp.int32, dist.shape, 1)
    lab_ref[...] = jnp.min(jnp.where(dist == mins, cidx, C), axis=1,
                           keepdims=True)


def _apply_kernel(x_ref, c_ref, sel_ref, lab_ref, o_ref):
    lab = lab_ref[...][None, :, :]
    b0 = (lab & 1) != 0
    b1 = (lab & 2) != 0
    b2 = (lab & 4) != 0
    c = c_ref[...]
    s0 = jnp.where(b0, c[:, 1:2, :], c[:, 0:1, :])
    s1 = jnp.where(b0, c[:, 3:4, :], c[:, 2:3, :])
    s2 = jnp.where(b0, c[:, 5:6, :], c[:, 4:5, :])
    t0 = jnp.where(b1, s1, s0)
    t1 = jnp.where(b1, c[:, 6:7, :], s2)
    keys = jnp.where(b2, t1, t0)
    o_ref[...] = x_ref[...] + sel_ref[...][None, :, :] * keys


def kernel(feats, centroids, W_sel, b_sel):
    B, D, H, W = feats.shape
    C = centroids.shape[0]
    HW = H * W
    x = jnp.transpose(feats, (2, 3, 0, 1)).reshape(HW, B, D)
    cents = jnp.transpose(centroids, (2, 3, 0, 1)).reshape(HW, C, D)
    b2 = b_sel.reshape(1, D)

    csum = pl.pallas_call(
        _csum_kernel,
        out_shape=jax.ShapeDtypeStruct((HW, C), jnp.float32),
    )(cents)

    BB = 8
    sel, labels = pl.pallas_call(
        _stats_kernel,
        grid=(B // BB,),
        in_specs=[
            pl.BlockSpec((HW, BB, D), lambda i: (0, i, 0)),
            pl.BlockSpec((HW, C), lambda i: (0, 0)),
            pl.BlockSpec((D, D), lambda i: (0, 0)),
            pl.BlockSpec((1, D), lambda i: (0, 0)),
        ],
        out_specs=[
            pl.BlockSpec((BB, D), lambda i: (i, 0)),
            pl.BlockSpec((BB, 1), lambda i: (i, 0)),
        ],
        out_shape=[
            jax.ShapeDtypeStruct((B, D), jnp.float32),
            jax.ShapeDtypeStruct((B, 1), jnp.int32),
        ],
        compiler_params=pltpu.CompilerParams(
            dimension_semantics=("parallel",),
        ),
    )(x, csum, W_sel, b2)

    HWB = 14
    out = pl.pallas_call(
        _apply_kernel,
        grid=(HW // HWB,),
        in_specs=[
            pl.BlockSpec((HWB, B, D), lambda j: (j, 0, 0)),
            pl.BlockSpec((HWB, C, D), lambda j: (j, 0, 0)),
            pl.BlockSpec((B, D), lambda j: (0, 0)),
            pl.BlockSpec((B, 1), lambda j: (0, 0)),
        ],
        out_specs=pl.BlockSpec((HWB, B, D), lambda j: (j, 0, 0)),
        out_shape=jax.ShapeDtypeStruct((HW, B, D), jnp.float32),
        compiler_params=pltpu.CompilerParams(
            dimension_semantics=("parallel",),
        ),
    )(x, cents, sel, labels)
    return jnp.transpose(out.reshape(H, W, B, D), (2, 3, 0, 1))


# speedup vs baseline: 2.5179x; 1.0107x over previous
import jax
import jax.numpy as jnp
from jax.experimental import pallas as pl
from jax.experimental.pallas import tpu as pltpu


def _csum_kernel(c_ref, o_ref):
    o_ref[...] = jnp.sum(c_ref[...], axis=2).T


def _stats_kernel(x_ref, cs_ref, w_ref, b_ref, sel_ref, lab_ref):
    C = cs_ref.shape[1]
    x = x_ref[...]

    pooled = jnp.mean(x, axis=0)
    sel_ref[...] = jnp.tanh(
        jax.lax.dot_general(
            pooled, w_ref[...], (((1,), (1,)), ((), ())),
            preferred_element_type=jnp.float32,
        )
        + b_ref[...]
    )

    fs = jnp.sum(x, axis=2, keepdims=True)
    cs = cs_ref[...][:, None, :]
    dist = jnp.sum(jnp.abs(fs - cs), axis=0)
    mins = jnp.min(dist, axis=1, keepdims=True)
    cidx = jax.lax.broadcasted_iota(jnp.int32, dist.shape, 1)
    lab_ref[...] = jnp.min(jnp.where(dist == mins, cidx, C), axis=1,
                           keepdims=True)


def _apply_kernel(x_ref, c_ref, sel_ref, lab_ref, o_ref):
    lab = lab_ref[...][None, :, :]
    b0 = (lab & 1) != 0
    b1 = (lab & 2) != 0
    b2 = (lab & 4) != 0
    c = c_ref[...]
    s0 = jnp.where(b0, c[:, 1:2, :], c[:, 0:1, :])
    s1 = jnp.where(b0, c[:, 3:4, :], c[:, 2:3, :])
    s2 = jnp.where(b0, c[:, 5:6, :], c[:, 4:5, :])
    t0 = jnp.where(b1, s1, s0)
    t1 = jnp.where(b1, c[:, 6:7, :], s2)
    keys = jnp.where(b2, t1, t0)
    o_ref[...] = x_ref[...] + sel_ref[...][None, :, :] * keys


def kernel(feats, centroids, W_sel, b_sel):
    B, D, H, W = feats.shape
    C = centroids.shape[0]
    HW = H * W
    x = jnp.transpose(feats, (2, 3, 0, 1)).reshape(HW, B, D)
    cents_native = jnp.transpose(centroids, (0, 2, 3, 1)).reshape(C, HW, D)
    cents = jnp.transpose(centroids, (2, 3, 0, 1)).reshape(HW, C, D)
    b2 = b_sel.reshape(1, D)

    csum = pl.pallas_call(
        _csum_kernel,
        out_shape=jax.ShapeDtypeStruct((HW, C), jnp.float32),
    )(cents_native)

    BB = 8
    sel, labels = pl.pallas_call(
        _stats_kernel,
        grid=(B // BB,),
        in_specs=[
            pl.BlockSpec((HW, BB, D), lambda i: (0, i, 0)),
            pl.BlockSpec((HW, C), lambda i: (0, 0)),
            pl.BlockSpec((D, D), lambda i: (0, 0)),
            pl.BlockSpec((1, D), lambda i: (0, 0)),
        ],
        out_specs=[
            pl.BlockSpec((BB, D), lambda i: (i, 0)),
            pl.BlockSpec((BB, 1), lambda i: (i, 0)),
        ],
        out_shape=[
            jax.ShapeDtypeStruct((B, D), jnp.float32),
            jax.ShapeDtypeStruct((B, 1), jnp.int32),
        ],
        compiler_params=pltpu.CompilerParams(
            dimension_semantics=("parallel",),
        ),
    )(x, csum, W_sel, b2)

    HWB = 14
    out = pl.pallas_call(
        _apply_kernel,
        grid=(HW // HWB,),
        in_specs=[
            pl.BlockSpec((HWB, B, D), lambda j: (j, 0, 0)),
            pl.BlockSpec((HWB, C, D), lambda j: (j, 0, 0)),
            pl.BlockSpec((B, D), lambda j: (0, 0)),
            pl.BlockSpec((B, 1), lambda j: (0, 0)),
        ],
        out_specs=pl.BlockSpec((HWB, B, D), lambda j: (j, 0, 0)),
        out_shape=jax.ShapeDtypeStruct((HW, B, D), jnp.float32),
        compiler_params=pltpu.CompilerParams(
            dimension_semantics=("parallel",),
        ),
    )(x, cents, sel, labels)
    return jnp.transpose(out.reshape(H, W, B, D), (2, 3, 0, 1))


# speedup vs baseline: 3.6795x; 1.4614x over previous
import jax
import jax.numpy as jnp
from jax.experimental import pallas as pl
from jax.experimental.pallas import tpu as pltpu


def _csum_kernel(c_ref, o_ref):
    o_ref[...] = jnp.sum(c_ref[...], axis=3)


def _unsup_embed_kernel(x_ref, c_ref, cs_ref, w_ref, b_ref, o_ref):
    C = c_ref.shape[1]
    HW, BB, D = x_ref.shape
    x = x_ref[...]

    pooled = jnp.mean(x, axis=0)
    sel = jnp.tanh(
        jax.lax.dot_general(
            pooled, w_ref[...], (((1,), (1,)), ((), ())),
            preferred_element_type=jnp.float32,
        )
        + b_ref[...]
    )

    fs = jnp.sum(x, axis=2, keepdims=True)
    cs = cs_ref[...][:, None, :]
    dist = jnp.sum(jnp.abs(fs - cs), axis=0)
    mins = jnp.min(dist, axis=1, keepdims=True)
    cidx = jax.lax.broadcasted_iota(jnp.int32, dist.shape, 1)
    first = jnp.min(jnp.where(dist == mins, cidx, C), axis=1, keepdims=True)

    idx = jnp.broadcast_to(first[None, :, :], (HW, BB, D))
    keys = jnp.take_along_axis(c_ref[...], idx, axis=1,
                               mode="promise_in_bounds")
    o_ref[...] = x + sel[None, :, :] * keys


def kernel(feats, centroids, W_sel, b_sel):
    B, D, H, W = feats.shape
    C = centroids.shape[0]
    HW = H * W
    x = jnp.transpose(feats, (2, 3, 0, 1)).reshape(HW, B, D)
    cents4 = jnp.transpose(centroids, (0, 2, 3, 1))
    cents = jnp.transpose(centroids, (2, 3, 0, 1)).reshape(HW, C, D)
    b2 = b_sel.reshape(1, D)

    csum4 = pl.pallas_call(
        _csum_kernel,
        out_shape=jax.ShapeDtypeStruct((C, H, W), jnp.float32),
    )(cents4)
    csum = csum4.reshape(C, HW).T

    BB = 8
    out = pl.pallas_call(
        _unsup_embed_kernel,
        grid=(B // BB,),
        in_specs=[
            pl.BlockSpec((HW, BB, D), lambda i: (0, i, 0)),
            pl.BlockSpec((HW, C, D), lambda i: (0, 0, 0)),
            pl.BlockSpec((HW, C), lambda i: (0, 0)),
            pl.BlockSpec((D, D), lambda i: (0, 0)),
            pl.BlockSpec((1, D), lambda i: (0, 0)),
        ],
        out_specs=pl.BlockSpec((HW, BB, D), lambda i: (0, i, 0)),
        out_shape=jax.ShapeDtypeStruct((HW, B, D), jnp.float32),
        compiler_params=pltpu.CompilerParams(
            dimension_semantics=("parallel",),
        ),
    )(x, cents, csum, W_sel, b2)
    return jnp.transpose(out.reshape(H, W, B, D), (2, 3, 0, 1))


# speedup vs baseline: 4.3372x; 1.1787x over previous
import jax
import jax.numpy as jnp
from jax.experimental import pallas as pl
from jax.experimental.pallas import tpu as pltpu


def _prep_kernel(c_ref, ct_ref, cs_ref):
    x = c_ref[...]
    cs_ref[...] = jnp.sum(x, axis=3)
    ct_ref[...] = jnp.transpose(x, (1, 2, 0, 3))


def _unsup_embed_kernel(x_ref, c_ref, cs_ref, w_ref, b_ref, o_ref):
    C = c_ref.shape[1]
    HW, BB, D = x_ref.shape
    x = x_ref[...]

    pooled = jnp.mean(x, axis=0)
    sel = jnp.tanh(
        jax.lax.dot_general(
            pooled, w_ref[...], (((1,), (1,)), ((), ())),
            preferred_element_type=jnp.float32,
        )
        + b_ref[...]
    )

    fs = jnp.sum(x, axis=2, keepdims=True)
    cs = cs_ref[...][:, None, :]
    dist = jnp.sum(jnp.abs(fs - cs), axis=0)
    mins = jnp.min(dist, axis=1, keepdims=True)
    cidx = jax.lax.broadcasted_iota(jnp.int32, dist.shape, 1)
    first = jnp.min(jnp.where(dist == mins, cidx, C), axis=1, keepdims=True)

    idx = jnp.broadcast_to(first[None, :, :], (HW, BB, D))
    keys = jnp.take_along_axis(c_ref[...], idx, axis=1,
                               mode="promise_in_bounds")
    o_ref[...] = x + sel[None, :, :] * keys


def kernel(feats, centroids, W_sel, b_sel):
    B, D, H, W = feats.shape
    C = centroids.shape[0]
    HW = H * W
    x = jnp.transpose(feats, (2, 3, 0, 1)).reshape(HW, B, D)
    cents4 = jnp.transpose(centroids, (0, 2, 3, 1))
    b2 = b_sel.reshape(1, D)

    cents_t4, csum4 = pl.pallas_call(
        _prep_kernel,
        out_shape=[
            jax.ShapeDtypeStruct((H, W, C, D), jnp.float32),
            jax.ShapeDtypeStruct((C, H, W), jnp.float32),
        ],
    )(cents4)
    cents = cents_t4.reshape(HW, C, D)
    csum = csum4.reshape(C, HW).T

    BB = 8
    out = pl.pallas_call(
        _unsup_embed_kernel,
        grid=(B // BB,),
        in_specs=[
            pl.BlockSpec((HW, BB, D), lambda i: (0, i, 0)),
            pl.BlockSpec((HW, C, D), lambda i: (0, 0, 0)),
            pl.BlockSpec((HW, C), lambda i: (0, 0)),
            pl.BlockSpec((D, D), lambda i: (0, 0)),
            pl.BlockSpec((1, D), lambda i: (0, 0)),
        ],
        out_specs=pl.BlockSpec((HW, BB, D), lambda i: (0, i, 0)),
        out_shape=jax.ShapeDtypeStruct((HW, B, D), jnp.float32),
        compiler_params=pltpu.CompilerParams(
            dimension_semantics=("parallel",),
        ),
    )(x, cents, csum, W_sel, b2)
    return jnp.transpose(out.reshape(H, W, B, D), (2, 3, 0, 1))


# speedup vs baseline: 4.3585x; 1.0049x over previous
import jax
import jax.numpy as jnp
from jax.experimental import pallas as pl
from jax.experimental.pallas import tpu as pltpu


def _prep_kernel(c_ref, ct_ref, cs_ref):
    x = c_ref[...]
    cs_ref[...] = jnp.transpose(jnp.sum(x, axis=3), (1, 2, 0))
    ct_ref[...] = jnp.transpose(x, (1, 2, 0, 3))


def _unsup_embed_kernel(x_ref, c_ref, cs_ref, w_ref, b_ref, o_ref):
    C = c_ref.shape[1]
    HW, BB, D = x_ref.shape
    x = x_ref[...]

    pooled = jnp.mean(x, axis=0)
    sel = jnp.tanh(
        jax.lax.dot_general(
            pooled, w_ref[...], (((1,), (1,)), ((), ())),
            preferred_element_type=jnp.float32,
        )
        + b_ref[...]
    )

    fs = jnp.sum(x, axis=2, keepdims=True)
    cs = cs_ref[...][:, None, :]
    dist = jnp.sum(jnp.abs(fs - cs), axis=0)
    mins = jnp.min(dist, axis=1, keepdims=True)
    cidx = jax.lax.broadcasted_iota(jnp.int32, dist.shape, 1)
    first = jnp.min(jnp.where(dist == mins, cidx, C), axis=1, keepdims=True)

    idx = jnp.broadcast_to(first[None, :, :], (HW, BB, D))
    keys = jnp.take_along_axis(c_ref[...], idx, axis=1,
                               mode="promise_in_bounds")
    o_ref[...] = x + sel[None, :, :] * keys


def kernel(feats, centroids, W_sel, b_sel):
    B, D, H, W = feats.shape
    C = centroids.shape[0]
    HW = H * W
    x = jnp.transpose(feats, (2, 3, 0, 1)).reshape(HW, B, D)
    cents4 = jnp.transpose(centroids, (0, 2, 3, 1))
    b2 = b_sel.reshape(1, D)

    cents_t4, csum4 = pl.pallas_call(
        _prep_kernel,
        out_shape=[
            jax.ShapeDtypeStruct((H, W, C, D), jnp.float32),
            jax.ShapeDtypeStruct((H, W, C), jnp.float32),
        ],
    )(cents4)
    cents = cents_t4.reshape(HW, C, D)
    csum = csum4.reshape(HW, C)

    BB = 8
    out = pl.pallas_call(
        _unsup_embed_kernel,
        grid=(B // BB,),
        in_specs=[
            pl.BlockSpec((HW, BB, D), lambda i: (0, i, 0)),
            pl.BlockSpec((HW, C, D), lambda i: (0, 0, 0)),
            pl.BlockSpec((HW, C), lambda i: (0, 0)),
            pl.BlockSpec((D, D), lambda i: (0, 0)),
            pl.BlockSpec((1, D), lambda i: (0, 0)),
        ],
        out_specs=pl.BlockSpec((HW, BB, D), lambda i: (0, i, 0)),
        out_shape=jax.ShapeDtypeStruct((HW, B, D), jnp.float32),
        compiler_params=pltpu.CompilerParams(
            dimension_semantics=("parallel",),
        ),
    )(x, cents, csum, W_sel, b2)
    return jnp.transpose(out.reshape(H, W, B, D), (2, 3, 0, 1))


# speedup vs baseline: 4.4377x; 1.0182x over previous
import jax
import jax.numpy as jnp
from jax.experimental import pallas as pl
from jax.experimental.pallas import tpu as pltpu


def _prep_kernel(c_ref, ct_ref, cs_ref):
    x = c_ref[...]
    HW = x.shape[1] * x.shape[2]
    cs_ref[...] = jnp.transpose(jnp.sum(x, axis=3), (1, 2, 0)).reshape(HW, x.shape[0])
    ct_ref[...] = jnp.transpose(x, (1, 2, 0, 3))


def _unsup_embed_kernel(x_ref, c_ref, cs_ref, w_ref, b_ref, o_ref):
    C = c_ref.shape[1]
    HW, BB, D = x_ref.shape
    x = x_ref[...]

    pooled = jnp.mean(x, axis=0)
    sel = jnp.tanh(
        jax.lax.dot_general(
            pooled, w_ref[...], (((1,), (1,)), ((), ())),
            preferred_element_type=jnp.float32,
        )
        + b_ref[...]
    )

    fs = jnp.sum(x, axis=2, keepdims=True)
    cs = cs_ref[...][:, None, :]
    dist = jnp.sum(jnp.abs(fs - cs), axis=0)
    mins = jnp.min(dist, axis=1, keepdims=True)
    cidx = jax.lax.broadcasted_iota(jnp.int32, dist.shape, 1)
    first = jnp.min(jnp.where(dist == mins, cidx, C), axis=1, keepdims=True)

    idx = jnp.broadcast_to(first[None, :, :], (HW, BB, D))
    keys = jnp.take_along_axis(c_ref[...], idx, axis=1,
                               mode="promise_in_bounds")
    o_ref[...] = x + sel[None, :, :] * keys


def kernel(feats, centroids, W_sel, b_sel):
    B, D, H, W = feats.shape
    C = centroids.shape[0]
    HW = H * W
    x = jnp.transpose(feats, (2, 3, 0, 1)).reshape(HW, B, D)
    cents4 = jnp.transpose(centroids, (0, 2, 3, 1))
    b2 = b_sel.reshape(1, D)

    cents_t4, csum4 = pl.pallas_call(
        _prep_kernel,
        out_shape=[
            jax.ShapeDtypeStruct((H, W, C, D), jnp.float32),
            jax.ShapeDtypeStruct((HW, C), jnp.float32),
        ],
    )(cents4)
    cents = cents_t4.reshape(HW, C, D)
    csum = csum4

    BB = 8
    out = pl.pallas_call(
        _unsup_embed_kernel,
        grid=(B // BB,),
        in_specs=[
            pl.BlockSpec((HW, BB, D), lambda i: (0, i, 0)),
            pl.BlockSpec((HW, C, D), lambda i: (0, 0, 0)),
            pl.BlockSpec((HW, C), lambda i: (0, 0)),
            pl.BlockSpec((D, D), lambda i: (0, 0)),
            pl.BlockSpec((1, D), lambda i: (0, 0)),
        ],
        out_specs=pl.BlockSpec((HW, BB, D), lambda i: (0, i, 0)),
        out_shape=jax.ShapeDtypeStruct((HW, B, D), jnp.float32),
        compiler_params=pltpu.CompilerParams(
            dimension_semantics=("parallel",),
        ),
    )(x, cents, csum, W_sel, b2)
    return jnp.transpose(out.reshape(H, W, B, D), (2, 3, 0, 1))


# speedup vs baseline: 4.7783x; 1.0767x over previous
import jax
import jax.numpy as jnp
from jax.experimental import pallas as pl
from jax.experimental.pallas import tpu as pltpu


def _unsup_embed_kernel(x_ref, c4_ref, w_ref, b_ref, o_ref, ct_ref, cs_ref):
    C = c4_ref.shape[0]
    HW, BB, D = x_ref.shape

    @pl.when(pl.program_id(0) == 0)
    def _build_tables():
        c4 = c4_ref[...]
        ct_ref[...] = jnp.transpose(c4, (1, 2, 0, 3)).reshape(HW, C, D)
        cs_ref[...] = (
            jnp.transpose(jnp.sum(c4, axis=3), (1, 2, 0)).reshape(HW, C)
        )

    x = x_ref[...]

    pooled = jnp.mean(x, axis=0)
    sel = jnp.tanh(
        jax.lax.dot_general(
            pooled, w_ref[...], (((1,), (1,)), ((), ())),
            preferred_element_type=jnp.float32,
        )
        + b_ref[...]
    )

    fs = jnp.sum(x, axis=2, keepdims=True)
    cs = cs_ref[...][:, None, :]
    dist = jnp.sum(jnp.abs(fs - cs), axis=0)
    mins = jnp.min(dist, axis=1, keepdims=True)
    cidx = jax.lax.broadcasted_iota(jnp.int32, dist.shape, 1)
    first = jnp.min(jnp.where(dist == mins, cidx, C), axis=1, keepdims=True)

    idx = jnp.broadcast_to(first[None, :, :], (HW, BB, D))
    keys = jnp.take_along_axis(ct_ref[...], idx, axis=1,
                               mode="promise_in_bounds")
    o_ref[...] = x + sel[None, :, :] * keys


def kernel(feats, centroids, W_sel, b_sel):
    B, D, H, W = feats.shape
    C = centroids.shape[0]
    HW = H * W
    x = jnp.transpose(feats, (2, 3, 0, 1)).reshape(HW, B, D)
    cents4 = jnp.transpose(centroids, (0, 2, 3, 1))
    b2 = b_sel.reshape(1, D)

    BB = 8
    out = pl.pallas_call(
        _unsup_embed_kernel,
        grid=(B // BB,),
        in_specs=[
            pl.BlockSpec((HW, BB, D), lambda i: (0, i, 0)),
            pl.BlockSpec((C, H, W, D), lambda i: (0, 0, 0, 0)),
            pl.BlockSpec((D, D), lambda i: (0, 0)),
            pl.BlockSpec((1, D), lambda i: (0, 0)),
        ],
        out_specs=pl.BlockSpec((HW, BB, D), lambda i: (0, i, 0)),
        out_shape=jax.ShapeDtypeStruct((HW, B, D), jnp.float32),
        scratch_shapes=[
            pltpu.VMEM((HW, C, D), jnp.float32),
            pltpu.VMEM((HW, C), jnp.float32),
        ],
        compiler_params=pltpu.CompilerParams(
            dimension_semantics=("arbitrary",),
        ),
    )(x, cents4, W_sel, b2)
    return jnp.transpose(out.reshape(H, W, B, D), (2, 3, 0, 1))
